# Initial kernel scaffold; baseline (speedup 1.0000x reference)
import jax
import jax.numpy as jnp
from jax.experimental import pallas as pl


def kernel(sim_matrix, b_ids, i_ids, j_ids):
    raise NotImplementedError("write your pallas kernel here")


# speedup vs baseline: 3.6903x; 3.6903x over previous
import functools

import numpy as np
import jax
import jax.numpy as jnp
from jax import lax
from jax.experimental import pallas as pl
from jax.experimental.pallas import tpu as pltpu
from jax.experimental.pallas import tpu_sc as plsc

_MARGIN = 1.0
_N_NEG = 10
_B, _L, _S = 4, 4096, 4096
_P = 8192
_LANES = 16
_NC, _NS = 2, 16
_NW = _NC * _NS
_RPW = _P // _NW
_NVEC = _S // _LANES
_CH = 8
_NCHUNK = _RPW // _CH
_NEGINF = -3.0e38
_MASKVAL = -1000000000.0
_SCALE = 1.0 / (_P * _N_NEG)

def _sortd(x):
    k, _ = plsc.sort_key_val(x, x, descending=True)
    return k


def _rev(x):
    return lax.rev(x, (0,))


def _body(table, rowidx, jids, mref, out, idx_v, jv, rowbuf, cand, accv, m_v):
    wid = lax.axis_index("s") * _NC + lax.axis_index("c")
    pltpu.sync_copy(rowidx.at[wid], idx_v)
    pltpu.sync_copy(jids.at[wid], jv.at[pl.ds(0, _RPW)])
    pltpu.sync_copy(mref, m_v)

    ninf = jnp.full((_LANES,), _NEGINF, jnp.float32)
    lanes = lax.iota(jnp.int32, _LANES)
    lane3 = lanes == 3
    m0c = m_v[0]
    m1c = m_v[1]

    def chunk_body(c, acc):
        pltpu.sync_copy(table.at[idx_v.at[c]], rowbuf)

        jvec = jv[pl.ds(c * _CH, _LANES)]

        def row_body(rl, acc):
            jj = lax.reduce_max(
                jnp.where(lanes == rl, jvec, 0), axes=(0,))
            grp = (jj // _LANES) * _LANES
            off = jj - grp
            va = rowbuf[rl, pl.ds(grp, _LANES)]
            pos = lax.reduce_max(
                jnp.where(lanes == off, va, _NEGINF), axes=(0,))
            rowbuf[rl, pl.ds(grp, _LANES)] = jnp.where(
                lanes == off, _MASKVAL, va)

            def p1(i, c2):
                e1, e2 = c2
                v = rowbuf[rl, pl.ds(i * _LANES, _LANES)]
                e2 = jnp.maximum(e2, jnp.minimum(e1, v))
                e1 = jnp.maximum(e1, v)
                return e1, e2

            e1, e2 = lax.fori_loop(0, _NVEC, p1, (ninf, ninf))
            lo = jnp.minimum(_sortd(e1), _rev(_sortd(e2)))
            lo_s = _sortd(lo)
            tau = lax.reduce_max(jnp.where(lane3, lo_s, _NEGINF), axes=(0,))

            def p2(i, cnt):
                v = rowbuf[rl, pl.ds(i * _LANES, _LANES)]
                m = v >= tau
                plsc.store_compressed(cand.at[pl.ds(cnt, _LANES)], v, mask=m)
                inc = lax.reduce_sum(jnp.where(m, 1, 0).astype(jnp.int32), axes=(0,))
                return cnt + inc

            cnt = lax.fori_loop(0, _NVEC, p2, jnp.int32(0))
            cand[pl.ds(cnt, _LANES)] = ninf
            nv = (cnt + _LANES - 1) // _LANES

            def mrg(t, c2):
                t0, t1 = c2
                sv = _sortd(cand[pl.ds(t * _LANES, _LANES)])
                a = _sortd(jnp.maximum(t1, _rev(sv)))
                ra = _rev(a)
                return _sortd(jnp.maximum(t0, ra)), _sortd(jnp.minimum(t0, ra))

            t0, t1 = lax.fori_loop(0, nv, mrg, (ninf, ninf))
            d = _MARGIN - pos
            contrib = (m0c * jnp.maximum(d + t0, 0.0)
                       + m1c * jnp.maximum(d + t1, 0.0))
            return acc + contrib

        return lax.fori_loop(0, _CH, row_body, acc)

    acc = lax.fori_loop(0, _NCHUNK, chunk_body, jnp.zeros((_LANES,), jnp.float32))
    accv[...] = acc * _SCALE
    pltpu.sync_copy(accv, out.at[wid])


_triplet_sc = functools.partial(
    pl.kernel,
    out_type=jax.ShapeDtypeStruct((_NW, _LANES), jnp.float32),
    mesh=plsc.VectorSubcoreMesh(
        core_axis_name="c", subcore_axis_name="s",
        num_cores=_NC, num_subcores=_NS),
    scratch_types=[
        pltpu.VMEM((_NCHUNK, _CH), jnp.int32),
        pltpu.VMEM((_RPW + _LANES,), jnp.int32),
        pltpu.VMEM((_CH, _S), jnp.float32),
        pltpu.VMEM((_S + _LANES,), jnp.float32),
        pltpu.VMEM((_LANES,), jnp.float32),
        pltpu.VMEM((2, _LANES), jnp.float32),
    ],
    compiler_params=pltpu.CompilerParams(needs_layout_passes=False),
)(_body)


def kernel(sim_matrix, b_ids, i_ids, j_ids):
    table = sim_matrix.reshape(_B * _L, _S)
    rowidx = (b_ids * _L + i_ids).astype(jnp.int32).reshape(_NW, _NCHUNK, _CH)
    jr = j_ids.astype(jnp.int32).reshape(_NW, _RPW)
    perm = jax.random.permutation(jax.random.key(42), 2 * _N_NEG)[:_N_NEG]
    masks = (jnp.arange(2 * _LANES)[None, :] == perm[:, None]).astype(
        jnp.float32).sum(axis=0).reshape(2, _LANES)
    out = _triplet_sc(table, rowidx, jr, masks)
    return jnp.sum(out)


# speedup vs baseline: 5.8423x; 1.5831x over previous
import functools

import jax
import jax.numpy as jnp
from jax import lax
from jax.experimental import pallas as pl
from jax.experimental.pallas import tpu as pltpu
from jax.experimental.pallas import tpu_sc as plsc

_MARGIN = 1.0
_N_NEG = 10
_B, _L, _S = 4, 4096, 4096
_P = 8192
_LANES = 16
_NC, _NS = 2, 16
_NW = _NC * _NS
_RPW = _P // _NW
_NVEC = _S // _LANES
_CH = 8
_NCHUNK = _RPW // _CH
_NEGINF = -3.0e38
_MASKVAL = -1000000000.0
_SCALE = 1.0 / (_P * _N_NEG)


def _sortd(x):
    k, _ = plsc.sort_key_val(x, x, descending=True)
    return k


def _rev(x):
    return lax.rev(x, (0,))


def _body(table, rowidx, jids, mref, out, idx_v, jv, rowbuf, cand, accv, m_v):
    wid = lax.axis_index("s") * _NC + lax.axis_index("c")
    pltpu.sync_copy(rowidx.at[wid], idx_v)
    pltpu.sync_copy(jids.at[wid], jv.at[pl.ds(0, _RPW)])
    pltpu.sync_copy(mref, m_v)

    ninf = jnp.full((_LANES,), _NEGINF, jnp.float32)
    izero = jnp.zeros((_LANES,), jnp.int32)
    ione = jnp.full((_LANES,), 1, jnp.int32)
    i16 = jnp.full((_LANES,), _LANES, jnp.int32)
    lanes = lax.iota(jnp.int32, _LANES)
    lane3 = lanes == 3
    row8 = lanes < _CH
    m0c = m_v[0]
    m1c = m_v[1]

    def chunk_body(c, acc):
        pltpu.sync_copy(table.at[idx_v.at[c]], rowbuf)
        jvec = jv[pl.ds(c * _CH, _LANES)]
        posv = plsc.load_gather(rowbuf, [lanes, jvec], mask=row8)
        plsc.store_scatter(rowbuf, [lanes, jvec],
                           jnp.full((_LANES,), _MASKVAL, jnp.float32),
                           mask=row8)

        for rl in range(_CH):
            def p1(i, c2):
                e1, e2 = c2
                v = rowbuf[rl, pl.ds(i * _LANES, _LANES)]
                e2 = jnp.maximum(e2, jnp.minimum(e1, v))
                e1 = jnp.maximum(e1, v)
                return e1, e2

            e1, e2 = lax.fori_loop(0, _NVEC, p1, (ninf, ninf), unroll=8)
            lo = jnp.minimum(_sortd(e1), _rev(_sortd(e2)))
            lo_s = _sortd(lo)
            tau = lax.reduce_max(jnp.where(lane3, lo_s, _NEGINF), axes=(0,))

            def p2(i, oidx):
                v = rowbuf[rl, pl.ds(i * _LANES, _LANES)]
                m = v >= tau
                plsc.store_scatter(cand, [oidx], v, mask=m)
                return oidx + jnp.where(m, i16, izero)

            oidx = lax.fori_loop(0, _NVEC, p2, lanes, unroll=8)
            cnt = lax.shift_right_logical(oidx - lanes, 4)
            nmax = lax.reduce_max(cnt, axes=(0,))

            def mrg(s, c2):
                t0, t1 = c2
                v = jnp.where(cnt > s, cand[pl.ds(s * _LANES, _LANES)], _NEGINF)
                sv = _sortd(v)
                a = _sortd(jnp.maximum(t1, _rev(sv)))
                ra = _rev(a)
                return _sortd(jnp.maximum(t0, ra)), _sortd(jnp.minimum(t0, ra))

            t0, t1 = lax.fori_loop(0, nmax, mrg, (ninf, ninf))

            pos = lax.reduce_max(jnp.where(lanes == rl, posv, _NEGINF),
                                 axes=(0,))
            d = _MARGIN - pos
            acc = acc + (m0c * jnp.maximum(d + t0, 0.0)
                         + m1c * jnp.maximum(d + t1, 0.0))
        return acc

    acc = lax.fori_loop(0, _NCHUNK, chunk_body, jnp.zeros((_LANES,), jnp.float32))
    accv[...] = acc * _SCALE
    pltpu.sync_copy(accv, out.at[wid])


_triplet_sc = functools.partial(
    pl.kernel,
    out_type=jax.ShapeDtypeStruct((_NW, _LANES), jnp.float32),
    mesh=plsc.VectorSubcoreMesh(
        core_axis_name="c", subcore_axis_name="s",
        num_cores=_NC, num_subcores=_NS),
    scratch_types=[
        pltpu.VMEM((_NCHUNK, _CH), jnp.int32),
        pltpu.VMEM((_RPW + _LANES,), jnp.int32),
        pltpu.VMEM((_CH, _S), jnp.float32),
        pltpu.VMEM((_S,), jnp.float32),
        pltpu.VMEM((_LANES,), jnp.float32),
        pltpu.VMEM((2, _LANES), jnp.float32),
    ],
    compiler_params=pltpu.CompilerParams(needs_layout_passes=False),
)(_body)


def kernel(sim_matrix, b_ids, i_ids, j_ids):
    table = sim_matrix.reshape(_B * _L, _S)
    rowidx = (b_ids * _L + i_ids).astype(jnp.int32).reshape(_NW, _NCHUNK, _CH)
    jr = j_ids.astype(jnp.int32).reshape(_NW, _RPW)
    perm = jax.random.permutation(jax.random.key(42), 2 * _N_NEG)[:_N_NEG]
    masks = (jnp.arange(2 * _LANES)[None, :] == perm[:, None]).astype(
        jnp.float32).sum(axis=0).reshape(2, _LANES)
    out = _triplet_sc(table, rowidx, jr, masks)
    return jnp.sum(out)


# speedup vs baseline: 16.8144x; 2.8780x over previous
import functools

import jax
import jax.numpy as jnp
from jax import lax
from jax.experimental import pallas as pl
from jax.experimental.pallas import tpu as pltpu
from jax.experimental.pallas import tpu_sc as plsc

_MARGIN = 1.0
_N_NEG = 10
_B, _L, _S = 4, 4096, 4096
_P = 8192
_LANES = 16
_NC, _NS = 2, 16
_NW = _NC * _NS
_RPW = _P // _NW
_NVEC = _S // _LANES
_CH = 8
_NCHUNK = _RPW // _CH
_NEGINF = -3.0e38
_MASKVAL = -1000000000.0
_SCALE = 1.0 / (_P * _N_NEG)


def _sortd(x):
    k, _ = plsc.sort_key_val(x, x, descending=True)
    return k


def _rev(x):
    return lax.rev(x, (0,))


def _body(table, rowidx, jids, mref, out, idx_v, jv, rowbuf, cand, accv, m_v):
    wid = lax.axis_index("s") * _NC + lax.axis_index("c")
    pltpu.sync_copy(rowidx.at[wid], idx_v)
    pltpu.sync_copy(jids.at[wid], jv.at[pl.ds(0, _RPW)])
    pltpu.sync_copy(mref, m_v)

    ninf = jnp.full((_LANES,), _NEGINF, jnp.float32)
    izero = jnp.zeros((_LANES,), jnp.int32)
    ione = jnp.full((_LANES,), 1, jnp.int32)
    i16 = jnp.full((_LANES,), _LANES, jnp.int32)
    lanes = lax.iota(jnp.int32, _LANES)
    lane3 = lanes == 3
    row8 = lanes < _CH
    m0c = m_v[0]
    m1c = m_v[1]

    def chunk_body(c, acc):
        pltpu.sync_copy(table.at[idx_v.at[c]], rowbuf)
        jvec = jv[pl.ds(c * _CH, _LANES)]
        posv = plsc.load_gather(rowbuf, [lanes, jvec], mask=row8)
        plsc.store_scatter(rowbuf, [lanes, jvec],
                           jnp.full((_LANES,), _MASKVAL, jnp.float32),
                           mask=row8)

        for rl in range(_CH):
            def p1(i, c2):
                e1, e2 = c2
                v = rowbuf[rl, pl.ds(i * _LANES, _LANES)]
                e2 = jnp.maximum(e2, jnp.minimum(e1, v))
                e1 = jnp.maximum(e1, v)
                return e1, e2

            e1, e2 = lax.fori_loop(0, _NVEC, p1, (ninf, ninf), unroll=8)
            lo = jnp.minimum(_sortd(e1), _rev(_sortd(e2)))
            lo_s = _sortd(lo)
            tau = lax.reduce_max(jnp.where(lane3, lo_s, _NEGINF), axes=(0,))

            @plsc.parallel_loop(0, _NVEC, unroll=8, carry=lanes)
            def oidx(i, oidx):
                v = rowbuf[rl, pl.ds(i * _LANES, _LANES)]
                m = v >= tau
                plsc.store_scatter(cand, [oidx], v, mask=m)
                return oidx + jnp.where(m, i16, izero)
            cnt = lax.shift_right_logical(oidx - lanes, 4)
            nmax = lax.reduce_max(cnt, axes=(0,))

            def mrg(s, c2):
                t0, t1 = c2
                v = jnp.where(cnt > s, cand[pl.ds(s * _LANES, _LANES)], _NEGINF)
                sv = _sortd(v)
                a = _sortd(jnp.maximum(t1, _rev(sv)))
                ra = _rev(a)
                return _sortd(jnp.maximum(t0, ra)), _sortd(jnp.minimum(t0, ra))

            t0, t1 = lax.fori_loop(0, nmax, mrg, (ninf, ninf))

            pos = lax.reduce_max(jnp.where(lanes == rl, posv, _NEGINF),
                                 axes=(0,))
            d = _MARGIN - pos
            acc = acc + (m0c * jnp.maximum(d + t0, 0.0)
                         + m1c * jnp.maximum(d + t1, 0.0))
        return acc

    acc = lax.fori_loop(0, _NCHUNK, chunk_body, jnp.zeros((_LANES,), jnp.float32))
    accv[...] = acc * _SCALE
    pltpu.sync_copy(accv, out.at[wid])


_triplet_sc = functools.partial(
    pl.kernel,
    out_type=jax.ShapeDtypeStruct((_NW, _LANES), jnp.float32),
    mesh=plsc.VectorSubcoreMesh(
        core_axis_name="c", subcore_axis_name="s",
        num_cores=_NC, num_subcores=_NS),
    scratch_types=[
        pltpu.VMEM((_NCHUNK, _CH), jnp.int32),
        pltpu.VMEM((_RPW + _LANES,), jnp.int32),
        pltpu.VMEM((_CH, _S), jnp.float32),
        pltpu.VMEM((_S,), jnp.float32),
        pltpu.VMEM((_LANES,), jnp.float32),
        pltpu.VMEM((2, _LANES), jnp.float32),
    ],
    compiler_params=pltpu.CompilerParams(needs_layout_passes=False),
)(_body)


def kernel(sim_matrix, b_ids, i_ids, j_ids):
    table = sim_matrix.reshape(_B * _L, _S)
    rowidx = (b_ids * _L + i_ids).astype(jnp.int32).reshape(_NW, _NCHUNK, _CH)
    jr = j_ids.astype(jnp.int32).reshape(_NW, _RPW)
    perm = jax.random.permutation(jax.random.key(42), 2 * _N_NEG)[:_N_NEG]
    masks = (jnp.arange(2 * _LANES)[None, :] == perm[:, None]).astype(
        jnp.float32).sum(axis=0).reshape(2, _LANES)
    out = _triplet_sc(table, rowidx, jr, masks)
    return jnp.sum(out)


# speedup vs baseline: 21.5651x; 1.2825x over previous
import functools

import jax
import jax.numpy as jnp
from jax import lax
from jax.experimental import pallas as pl
from jax.experimental.pallas import tpu as pltpu
from jax.experimental.pallas import tpu_sc as plsc

_MARGIN = 1.0
_N_NEG = 10
_B, _L, _S = 4, 4096, 4096
_P = 8192
_LANES = 16
_NC, _NS = 2, 16
_NW = _NC * _NS
_RPW = _P // _NW
_NVEC = _S // _LANES
_CH = 8
_NCHUNK = _RPW // _CH
_NEGINF = -3.0e38
_MASKVAL = -1000000000.0
_SCALE = 1.0 / (_P * _N_NEG)


def _sortd(x):
    k, _ = plsc.sort_key_val(x, x, descending=True)
    return k


def _rev(x):
    return lax.rev(x, (0,))


def _body(table, rowidx, jids, mref, out, idx_v, jv, rowbuf_a, rowbuf_b,
          cand, accv, m_v, sem_a, sem_b):
    wid = lax.axis_index("s") * _NC + lax.axis_index("c")
    pltpu.sync_copy(rowidx.at[wid], idx_v)
    pltpu.sync_copy(jids.at[wid], jv.at[pl.ds(0, _RPW)])
    pltpu.sync_copy(mref, m_v)

    ninf = jnp.full((_LANES,), _NEGINF, jnp.float32)
    izero = jnp.zeros((_LANES,), jnp.int32)
    ione = jnp.full((_LANES,), 1, jnp.int32)
    i16 = jnp.full((_LANES,), _LANES, jnp.int32)
    lanes = lax.iota(jnp.int32, _LANES)
    lane3 = lanes == 3
    row8 = lanes < _CH
    m0c = m_v[0]
    m1c = m_v[1]

    def process_chunk(rowbuf, c, acc):
        jvec = jv[pl.ds(c * _CH, _LANES)]
        posv = plsc.load_gather(rowbuf, [lanes, jvec], mask=row8)
        plsc.store_scatter(rowbuf, [lanes, jvec],
                           jnp.full((_LANES,), _MASKVAL, jnp.float32),
                           mask=row8)

        for rl in range(_CH):
            def p1(i, c2):
                e1, e2 = c2
                v = rowbuf[rl, pl.ds(i * _LANES, _LANES)]
                e2 = jnp.maximum(e2, jnp.minimum(e1, v))
                e1 = jnp.maximum(e1, v)
                return e1, e2

            e1, e2 = lax.fori_loop(0, _NVEC, p1, (ninf, ninf), unroll=8)
            lo = jnp.minimum(_sortd(e1), _rev(_sortd(e2)))
            lo_s = _sortd(lo)
            tau = lax.reduce_max(jnp.where(lane3, lo_s, _NEGINF), axes=(0,))

            @plsc.parallel_loop(0, _NVEC, unroll=8, carry=lanes)
            def oidx(i, oidx):
                v = rowbuf[rl, pl.ds(i * _LANES, _LANES)]
                m = v >= tau
                plsc.store_scatter(cand, [oidx], v, mask=m)
                return oidx + jnp.where(m, i16, izero)
            cnt = lax.shift_right_logical(oidx - lanes, 4)
            nmax = lax.reduce_max(cnt, axes=(0,))

            def mrg(s, c2):
                t0, t1 = c2
                v = jnp.where(cnt > s, cand[pl.ds(s * _LANES, _LANES)], _NEGINF)
                sv = _sortd(v)
                a = _sortd(jnp.maximum(t1, _rev(sv)))
                ra = _rev(a)
                return _sortd(jnp.maximum(t0, ra)), _sortd(jnp.minimum(t0, ra))

            t0, t1 = lax.fori_loop(0, nmax, mrg, (ninf, ninf))

            pos = lax.reduce_max(jnp.where(lanes == rl, posv, _NEGINF),
                                 axes=(0,))
            d = _MARGIN - pos
            acc = acc + (m0c * jnp.maximum(d + t0, 0.0)
                         + m1c * jnp.maximum(d + t1, 0.0))
        return acc

    pltpu.async_copy(table.at[idx_v.at[0]], rowbuf_a, sem_a)
    pltpu.async_copy(table.at[idx_v.at[1]], rowbuf_b, sem_b)

    def pair_body(cc, acc):
        c0 = cc * 2
        c1 = c0 + 1
        pltpu.make_async_copy(table.at[idx_v.at[c0]], rowbuf_a, sem_a).wait()
        acc = process_chunk(rowbuf_a, c0, acc)

        @pl.when(c0 + 2 < _NCHUNK)
        def _():
            pltpu.async_copy(table.at[idx_v.at[c0 + 2]], rowbuf_a, sem_a)

        pltpu.make_async_copy(table.at[idx_v.at[c1]], rowbuf_b, sem_b).wait()
        acc = process_chunk(rowbuf_b, c1, acc)

        @pl.when(c1 + 2 < _NCHUNK)
        def _():
            pltpu.async_copy(table.at[idx_v.at[c1 + 2]], rowbuf_b, sem_b)

        return acc

    acc = lax.fori_loop(0, _NCHUNK // 2, pair_body,
                        jnp.zeros((_LANES,), jnp.float32))
    accv[...] = acc * _SCALE
    pltpu.sync_copy(accv, out.at[wid])


_triplet_sc = functools.partial(
    pl.kernel,
    out_type=jax.ShapeDtypeStruct((_NW, _LANES), jnp.float32),
    mesh=plsc.VectorSubcoreMesh(
        core_axis_name="c", subcore_axis_name="s",
        num_cores=_NC, num_subcores=_NS),
    scratch_types=[
        pltpu.VMEM((_NCHUNK, _CH), jnp.int32),
        pltpu.VMEM((_RPW + _LANES,), jnp.int32),
        pltpu.VMEM((_CH, _S), jnp.float32),
        pltpu.VMEM((_CH, _S), jnp.float32),
        pltpu.VMEM((_S,), jnp.float32),
        pltpu.VMEM((_LANES,), jnp.float32),
        pltpu.VMEM((2, _LANES), jnp.float32),
        pltpu.SemaphoreType.DMA,
        pltpu.SemaphoreType.DMA,
    ],
    compiler_params=pltpu.CompilerParams(needs_layout_passes=False),
)(_body)


def kernel(sim_matrix, b_ids, i_ids, j_ids):
    table = sim_matrix.reshape(_B * _L, _S)
    rowidx = (b_ids * _L + i_ids).astype(jnp.int32).reshape(_NW, _NCHUNK, _CH)
    jr = j_ids.astype(jnp.int32).reshape(_NW, _RPW)
    perm = jax.random.permutation(jax.random.key(42), 2 * _N_NEG)[:_N_NEG]
    masks = (jnp.arange(2 * _LANES)[None, :] == perm[:, None]).astype(
        jnp.float32).sum(axis=0).reshape(2, _LANES)
    out = _triplet_sc(table, rowidx, jr, masks)
    return jnp.sum(out)


# speedup vs baseline: 23.7592x; 1.1017x over previous
import functools

import jax
import jax.numpy as jnp
from jax import lax
from jax.experimental import pallas as pl
from jax.experimental.pallas import tpu as pltpu
from jax.experimental.pallas import tpu_sc as plsc

_MARGIN = 1.0
_N_NEG = 10
_B, _L, _S = 4, 4096, 4096
_P = 8192
_LANES = 16
_NC, _NS = 2, 16
_NW = _NC * _NS
_RPW = _P // _NW
_NVEC = _S // _LANES
_CH = 8
_NCHUNK = _RPW // _CH
_NEGINF = -3.0e38
_MASKVAL = -1000000000.0
_SCALE = 1.0 / (_P * _N_NEG)


def _sortd(x):
    k, _ = plsc.sort_key_val(x, x, descending=True)
    return k


def _rev(x):
    return lax.rev(x, (0,))


def _body(table, rowidx, jids, mref, out, idx_v, jv, rowbuf_a, rowbuf_b,
          cand, cand_b, accv, m_v, sem_a, sem_b):
    wid = lax.axis_index("s") * _NC + lax.axis_index("c")
    pltpu.sync_copy(rowidx.at[wid], idx_v)
    pltpu.sync_copy(jids.at[wid], jv.at[pl.ds(0, _RPW)])
    pltpu.sync_copy(mref, m_v)

    ninf = jnp.full((_LANES,), _NEGINF, jnp.float32)
    izero = jnp.zeros((_LANES,), jnp.int32)
    ione = jnp.full((_LANES,), 1, jnp.int32)
    i16 = jnp.full((_LANES,), _LANES, jnp.int32)
    lanes = lax.iota(jnp.int32, _LANES)
    lane3 = lanes == 3
    row8 = lanes < _CH
    m0c = m_v[0]
    m1c = m_v[1]

    def process_chunk(rowbuf, c, acc):
        jvec = jv[pl.ds(c * _CH, _LANES)]
        posv = plsc.load_gather(rowbuf, [lanes, jvec], mask=row8)
        plsc.store_scatter(rowbuf, [lanes, jvec],
                           jnp.full((_LANES,), _MASKVAL, jnp.float32),
                           mask=row8)

        def pass1(rl):
            def p1(i, c2):
                e1, e2 = c2
                v = rowbuf[rl, pl.ds(i * _LANES, _LANES)]
                e2 = jnp.maximum(e2, jnp.minimum(e1, v))
                e1 = jnp.maximum(e1, v)
                return e1, e2

            return lax.fori_loop(0, _NVEC, p1, (ninf, ninf), unroll=8)

        def tau_of(e1, e2):
            lo = jnp.minimum(_sortd(e1), _rev(_sortd(e2)))
            return lax.reduce_max(jnp.where(lane3, _sortd(lo), _NEGINF),
                                  axes=(0,))

        def pass2(rl, tau, cbuf):
            @plsc.parallel_loop(0, _NVEC, unroll=8, carry=lanes)
            def oidx(i, oidx):
                v = rowbuf[rl, pl.ds(i * _LANES, _LANES)]
                m = v >= tau
                plsc.store_scatter(cbuf, [oidx], v, mask=m)
                return oidx + jnp.where(m, i16, izero)

            return lax.shift_right_logical(oidx - lanes, 4)

        def merge1(s, t0, t1, cnt, cbuf):
            v = jnp.where(cnt > s, cbuf[pl.ds(s * _LANES, _LANES)], _NEGINF)
            sv = _sortd(v)
            a = _sortd(jnp.maximum(t1, _rev(sv)))
            ra = _rev(a)
            return _sortd(jnp.maximum(t0, ra)), _sortd(jnp.minimum(t0, ra))

        def hinge(rl, t0, t1):
            pos = lax.reduce_max(jnp.where(lanes == rl, posv, _NEGINF),
                                 axes=(0,))
            d = _MARGIN - pos
            return (m0c * jnp.maximum(d + t0, 0.0)
                    + m1c * jnp.maximum(d + t1, 0.0))

        for rp in range(_CH // 2):
            ra, rb = 2 * rp, 2 * rp + 1
            e1a, e2a = pass1(ra)
            e1b, e2b = pass1(rb)
            tau_a = tau_of(e1a, e2a)
            tau_b = tau_of(e1b, e2b)
            cnt_a = pass2(ra, tau_a, cand)
            cnt_b = pass2(rb, tau_b, cand_b)
            nmax = lax.reduce_max(jnp.maximum(cnt_a, cnt_b), axes=(0,))

            def mrg2(s, c4):
                t0a, t1a, t0b, t1b = c4
                t0a, t1a = merge1(s, t0a, t1a, cnt_a, cand)
                t0b, t1b = merge1(s, t0b, t1b, cnt_b, cand_b)
                return t0a, t1a, t0b, t1b

            t0a, t1a, t0b, t1b = lax.fori_loop(
                0, nmax, mrg2, (ninf, ninf, ninf, ninf))
            acc = acc + hinge(ra, t0a, t1a) + hinge(rb, t0b, t1b)
        return acc

    pltpu.async_copy(table.at[idx_v.at[0]], rowbuf_a, sem_a)
    pltpu.async_copy(table.at[idx_v.at[1]], rowbuf_b, sem_b)

    def pair_body(cc, acc):
        c0 = cc * 2
        c1 = c0 + 1
        pltpu.make_async_copy(table.at[idx_v.at[c0]], rowbuf_a, sem_a).wait()
        acc = process_chunk(rowbuf_a, c0, acc)

        @pl.when(c0 + 2 < _NCHUNK)
        def _():
            pltpu.async_copy(table.at[idx_v.at[c0 + 2]], rowbuf_a, sem_a)

        pltpu.make_async_copy(table.at[idx_v.at[c1]], rowbuf_b, sem_b).wait()
        acc = process_chunk(rowbuf_b, c1, acc)

        @pl.when(c1 + 2 < _NCHUNK)
        def _():
            pltpu.async_copy(table.at[idx_v.at[c1 + 2]], rowbuf_b, sem_b)

        return acc

    acc = lax.fori_loop(0, _NCHUNK // 2, pair_body,
                        jnp.zeros((_LANES,), jnp.float32))
    accv[...] = acc * _SCALE
    pltpu.sync_copy(accv, out.at[wid])


_triplet_sc = functools.partial(
    pl.kernel,
    out_type=jax.ShapeDtypeStruct((_NW, _LANES), jnp.float32),
    mesh=plsc.VectorSubcoreMesh(
        core_axis_name="c", subcore_axis_name="s",
        num_cores=_NC, num_subcores=_NS),
    scratch_types=[
        pltpu.VMEM((_NCHUNK, _CH), jnp.int32),
        pltpu.VMEM((_RPW + _LANES,), jnp.int32),
        pltpu.VMEM((_CH, _S), jnp.float32),
        pltpu.VMEM((_CH, _S), jnp.float32),
        pltpu.VMEM((_S,), jnp.float32),
        pltpu.VMEM((_S,), jnp.float32),
        pltpu.VMEM((_LANES,), jnp.float32),
        pltpu.VMEM((2, _LANES), jnp.float32),
        pltpu.SemaphoreType.DMA,
        pltpu.SemaphoreType.DMA,
    ],
    compiler_params=pltpu.CompilerParams(needs_layout_passes=False),
)(_body)


def kernel(sim_matrix, b_ids, i_ids, j_ids):
    table = sim_matrix.reshape(_B * _L, _S)
    rowidx = (b_ids * _L + i_ids).astype(jnp.int32).reshape(_NW, _NCHUNK, _CH)
    jr = j_ids.astype(jnp.int32).reshape(_NW, _RPW)
    perm = jax.random.permutation(jax.random.key(42), 2 * _N_NEG)[:_N_NEG]
    masks = (jnp.arange(2 * _LANES)[None, :] == perm[:, None]).astype(
        jnp.float32).sum(axis=0).reshape(2, _LANES)
    out = _triplet_sc(table, rowidx, jr, masks)
    return jnp.sum(out)


# speedup vs baseline: 24.0618x; 1.0127x over previous
import functools

import jax
import jax.numpy as jnp
from jax import lax
from jax.experimental import pallas as pl
from jax.experimental.pallas import tpu as pltpu
from jax.experimental.pallas import tpu_sc as plsc

_MARGIN = 1.0
_N_NEG = 10
_B, _L, _S = 4, 4096, 4096
_P = 8192
_LANES = 16
_NC, _NS = 2, 16
_NW = _NC * _NS
_RPW = _P // _NW
_NVEC = _S // _LANES
_CH = 8
_NCHUNK = _RPW // _CH
_NEGINF = -3.0e38
_MASKVAL = -1000000000.0
_SCALE = 1.0 / (_P * _N_NEG)


def _sortd(x):
    k, _ = plsc.sort_key_val(x, x, descending=True)
    return k


def _rev(x):
    return lax.rev(x, (0,))


def _body(table, rowidx, jids, mref, out, idx_v, jv, rowbuf_a, rowbuf_b,
          cand, cand_b, accv, m_v, sem_a, sem_b):
    wid = lax.axis_index("s") * _NC + lax.axis_index("c")
    pltpu.sync_copy(rowidx.at[wid], idx_v)
    pltpu.sync_copy(jids.at[wid], jv.at[pl.ds(0, _RPW)])
    pltpu.sync_copy(mref, m_v)

    ninf = jnp.full((_LANES,), _NEGINF, jnp.float32)
    izero = jnp.zeros((_LANES,), jnp.int32)
    ione = jnp.full((_LANES,), 1, jnp.int32)
    i16 = jnp.full((_LANES,), _LANES, jnp.int32)
    lanes = lax.iota(jnp.int32, _LANES)
    lane3 = lanes == 3
    row8 = lanes < _CH
    m0c = m_v[0]
    m1c = m_v[1]

    def process_chunk(rowbuf, c, acc):
        jvec = jv[pl.ds(c * _CH, _LANES)]
        posv = plsc.load_gather(rowbuf, [lanes, jvec], mask=row8)
        plsc.store_scatter(rowbuf, [lanes, jvec],
                           jnp.full((_LANES,), _MASKVAL, jnp.float32),
                           mask=row8)

        def pass1(rl):
            def p1(i, c2):
                e1, e2 = c2
                v = rowbuf[rl, pl.ds(i * _LANES, _LANES)]
                e2 = jnp.maximum(e2, jnp.minimum(e1, v))
                e1 = jnp.maximum(e1, v)
                return e1, e2

            return lax.fori_loop(0, _NVEC, p1, (ninf, ninf), unroll=16)

        def tau_of(e1, e2):
            lo = jnp.minimum(_sortd(e1), _rev(_sortd(e2)))
            return lax.reduce_max(jnp.where(lane3, _sortd(lo), _NEGINF),
                                  axes=(0,))

        def pass2(rl, tau, cbuf):
            @plsc.parallel_loop(0, _NVEC, unroll=16, carry=lanes)
            def oidx(i, oidx):
                v = rowbuf[rl, pl.ds(i * _LANES, _LANES)]
                m = v >= tau
                plsc.store_scatter(cbuf, [oidx], v, mask=m)
                return oidx + jnp.where(m, i16, izero)

            return lax.shift_right_logical(oidx - lanes, 4)

        def merge1(s, t0, t1, cnt, cbuf):
            v = jnp.where(cnt > s, cbuf[pl.ds(s * _LANES, _LANES)], _NEGINF)
            sv = _sortd(v)
            a = _sortd(jnp.maximum(t1, _rev(sv)))
            ra = _rev(a)
            return _sortd(jnp.maximum(t0, ra)), _sortd(jnp.minimum(t0, ra))

        def hinge(rl, t0, t1):
            pos = lax.reduce_max(jnp.where(lanes == rl, posv, _NEGINF),
                                 axes=(0,))
            d = _MARGIN - pos
            return (m0c * jnp.maximum(d + t0, 0.0)
                    + m1c * jnp.maximum(d + t1, 0.0))

        for rp in range(_CH // 2):
            ra, rb = 2 * rp, 2 * rp + 1
            e1a, e2a = pass1(ra)
            e1b, e2b = pass1(rb)
            tau_a = tau_of(e1a, e2a)
            tau_b = tau_of(e1b, e2b)
            cnt_a = pass2(ra, tau_a, cand)
            cnt_b = pass2(rb, tau_b, cand_b)
            nmax = lax.reduce_max(jnp.maximum(cnt_a, cnt_b), axes=(0,))

            def mrg2(s, c4):
                t0a, t1a, t0b, t1b = c4
                t0a, t1a = merge1(s, t0a, t1a, cnt_a, cand)
                t0b, t1b = merge1(s, t0b, t1b, cnt_b, cand_b)
                return t0a, t1a, t0b, t1b

            t0a, t1a, t0b, t1b = lax.fori_loop(
                0, nmax, mrg2, (ninf, ninf, ninf, ninf))
            acc = acc + hinge(ra, t0a, t1a) + hinge(rb, t0b, t1b)
        return acc

    pltpu.async_copy(table.at[idx_v.at[0]], rowbuf_a, sem_a)
    pltpu.async_copy(table.at[idx_v.at[1]], rowbuf_b, sem_b)

    def pair_body(cc, acc):
        c0 = cc * 2
        c1 = c0 + 1
        pltpu.make_async_copy(table.at[idx_v.at[c0]], rowbuf_a, sem_a).wait()
        acc = process_chunk(rowbuf_a, c0, acc)

        @pl.when(c0 + 2 < _NCHUNK)
        def _():
            pltpu.async_copy(table.at[idx_v.at[c0 + 2]], rowbuf_a, sem_a)

        pltpu.make_async_copy(table.at[idx_v.at[c1]], rowbuf_b, sem_b).wait()
        acc = process_chunk(rowbuf_b, c1, acc)

        @pl.when(c1 + 2 < _NCHUNK)
        def _():
            pltpu.async_copy(table.at[idx_v.at[c1 + 2]], rowbuf_b, sem_b)

        return acc

    acc = lax.fori_loop(0, _NCHUNK // 2, pair_body,
                        jnp.zeros((_LANES,), jnp.float32))
    accv[...] = acc * _SCALE
    pltpu.sync_copy(accv, out.at[wid])


_triplet_sc = functools.partial(
    pl.kernel,
    out_type=jax.ShapeDtypeStruct((_NW, _LANES), jnp.float32),
    mesh=plsc.VectorSubcoreMesh(
        core_axis_name="c", subcore_axis_name="s",
        num_cores=_NC, num_subcores=_NS),
    scratch_types=[
        pltpu.VMEM((_NCHUNK, _CH), jnp.int32),
        pltpu.VMEM((_RPW + _LANES,), jnp.int32),
        pltpu.VMEM((_CH, _S), jnp.float32),
        pltpu.VMEM((_CH, _S), jnp.float32),
        pltpu.VMEM((_S,), jnp.float32),
        pltpu.VMEM((_S,), jnp.float32),
        pltpu.VMEM((_LANES,), jnp.float32),
        pltpu.VMEM((2, _LANES), jnp.float32),
        pltpu.SemaphoreType.DMA,
        pltpu.SemaphoreType.DMA,
    ],
    compiler_params=pltpu.CompilerParams(needs_layout_passes=False),
)(_body)


def kernel(sim_matrix, b_ids, i_ids, j_ids):
    table = sim_matrix.reshape(_B * _L, _S)
    rowidx = (b_ids * _L + i_ids).astype(jnp.int32).reshape(_NW, _NCHUNK, _CH)
    jr = j_ids.astype(jnp.int32).reshape(_NW, _RPW)
    perm = jax.random.permutation(jax.random.key(42), 2 * _N_NEG)[:_N_NEG]
    masks = (jnp.arange(2 * _LANES)[None, :] == perm[:, None]).astype(
        jnp.float32).sum(axis=0).reshape(2, _LANES)
    out = _triplet_sc(table, rowidx, jr, masks)
    return jnp.sum(out)


# speedup vs baseline: 27.0116x; 1.1226x over previous
import functools

import jax
import jax.numpy as jnp
from jax import lax
from jax.experimental import pallas as pl
from jax.experimental.pallas import tpu as pltpu
from jax.experimental.pallas import tpu_sc as plsc

_MARGIN = 1.0
_N_NEG = 10
_B, _L, _S = 4, 4096, 4096
_P = 8192
_LANES = 16
_NC, _NS = 2, 16
_NW = _NC * _NS
_RPW = _P // _NW
_NVEC = _S // _LANES
_CH = 8
_NCHUNK = _RPW // _CH
_NEGINF = -3.0e38
_MASKVAL = -1000000000.0
_SCALE = 1.0 / (_P * _N_NEG)


def _sortd(x):
    k, _ = plsc.sort_key_val(x, x, descending=True)
    return k


def _rev(x):
    return lax.rev(x, (0,))


def _body(table, rowidx, jids, mref, out, idx_v, jv, rowbuf_a, rowbuf_b,
          cand, accv, m_v, sem_a, sem_b):
    wid = lax.axis_index("s") * _NC + lax.axis_index("c")
    pltpu.sync_copy(rowidx.at[wid], idx_v)
    pltpu.sync_copy(jids.at[wid], jv.at[pl.ds(0, _RPW)])
    pltpu.sync_copy(mref, m_v)

    ninf = jnp.full((_LANES,), _NEGINF, jnp.float32)
    izero = jnp.zeros((_LANES,), jnp.int32)
    ione = jnp.full((_LANES,), 1, jnp.int32)
    i16 = jnp.full((_LANES,), _LANES, jnp.int32)
    lanes = lax.iota(jnp.int32, _LANES)
    lane3 = lanes == 3
    row8 = lanes < _CH
    m0c = m_v[0]
    m1c = m_v[1]

    def process_chunk(rowbuf, c, acc):
        jvec = jv[pl.ds(c * _CH, _LANES)]
        posv = plsc.load_gather(rowbuf, [lanes, jvec], mask=row8)
        plsc.store_scatter(rowbuf, [lanes, jvec],
                           jnp.full((_LANES,), _MASKVAL, jnp.float32),
                           mask=row8)

        def pass1(rl):
            def p1(i, c2):
                e1, e2 = c2
                v = rowbuf[rl, pl.ds(i * _LANES, _LANES)]
                e2 = jnp.maximum(e2, jnp.minimum(e1, v))
                e1 = jnp.maximum(e1, v)
                return e1, e2

            return lax.fori_loop(0, _NVEC, p1, (ninf, ninf), unroll=16)

        def tau_of(e1, e2):
            lo = jnp.minimum(_sortd(e1), _rev(_sortd(e2)))
            return lax.reduce_max(jnp.where(lane3, _sortd(lo), _NEGINF),
                                  axes=(0,))

        def pass2(rl, tau):
            base = lanes + rl * _S

            @plsc.parallel_loop(0, _NVEC, unroll=16, carry=base)
            def oidx(i, oidx):
                v = rowbuf[rl, pl.ds(i * _LANES, _LANES)]
                m = v >= tau
                plsc.store_scatter(cand, [oidx], v, mask=m)
                return oidx + jnp.where(m, i16, izero)

            return lax.shift_right_logical(oidx - base, 4)

        def merge1(s, rl, t0, t1, cnt):
            v = jnp.where(cnt > s, cand[pl.ds(rl * _S + s * _LANES, _LANES)],
                          _NEGINF)
            sv = _sortd(v)
            a = _sortd(jnp.maximum(t1, _rev(sv)))
            ra = _rev(a)
            return _sortd(jnp.maximum(t0, ra)), _sortd(jnp.minimum(t0, ra))

        def hinge(rl, t0, t1):
            d = _MARGIN - posv[rl]
            return (m0c * jnp.maximum(d + t0, 0.0)
                    + m1c * jnp.maximum(d + t1, 0.0))

        es = [pass1(rl) for rl in range(_CH)]
        taus = [tau_of(e1, e2) for (e1, e2) in es]
        cnts = [pass2(rl, taus[rl]) for rl in range(_CH)]
        cmax = cnts[0]
        for rl in range(1, _CH):
            cmax = jnp.maximum(cmax, cnts[rl])
        nmax = lax.reduce_max(cmax, axes=(0,))

        def mrg8(s, ts):
            return tuple(
                v for rl in range(_CH)
                for v in merge1(s, rl, ts[2 * rl], ts[2 * rl + 1], cnts[rl]))

        ts = lax.fori_loop(0, nmax, mrg8, (ninf,) * (2 * _CH))
        for rl in range(_CH):
            acc = acc + hinge(rl, ts[2 * rl], ts[2 * rl + 1])
        return acc

    pltpu.async_copy(table.at[idx_v.at[0]], rowbuf_a, sem_a)
    pltpu.async_copy(table.at[idx_v.at[1]], rowbuf_b, sem_b)

    def pair_body(cc, acc):
        c0 = cc * 2
        c1 = c0 + 1
        pltpu.make_async_copy(table.at[idx_v.at[c0]], rowbuf_a, sem_a).wait()
        acc = process_chunk(rowbuf_a, c0, acc)

        @pl.when(c0 + 2 < _NCHUNK)
        def _():
            pltpu.async_copy(table.at[idx_v.at[c0 + 2]], rowbuf_a, sem_a)

        pltpu.make_async_copy(table.at[idx_v.at[c1]], rowbuf_b, sem_b).wait()
        acc = process_chunk(rowbuf_b, c1, acc)

        @pl.when(c1 + 2 < _NCHUNK)
        def _():
            pltpu.async_copy(table.at[idx_v.at[c1 + 2]], rowbuf_b, sem_b)

        return acc

    acc = lax.fori_loop(0, _NCHUNK // 2, pair_body,
                        jnp.zeros((_LANES,), jnp.float32))
    accv[...] = acc * _SCALE
    pltpu.sync_copy(accv, out.at[wid])


_triplet_sc = functools.partial(
    pl.kernel,
    out_type=jax.ShapeDtypeStruct((_NW, _LANES), jnp.float32),
    mesh=plsc.VectorSubcoreMesh(
        core_axis_name="c", subcore_axis_name="s",
        num_cores=_NC, num_subcores=_NS),
    scratch_types=[
        pltpu.VMEM((_NCHUNK, _CH), jnp.int32),
        pltpu.VMEM((_RPW + _LANES,), jnp.int32),
        pltpu.VMEM((_CH, _S), jnp.float32),
        pltpu.VMEM((_CH, _S), jnp.float32),
        pltpu.VMEM((_CH * _S,), jnp.float32),
        pltpu.VMEM((_LANES,), jnp.float32),
        pltpu.VMEM((2, _LANES), jnp.float32),
        pltpu.SemaphoreType.DMA,
        pltpu.SemaphoreType.DMA,
    ],
    compiler_params=pltpu.CompilerParams(needs_layout_passes=False),
)(_body)


def kernel(sim_matrix, b_ids, i_ids, j_ids):
    table = sim_matrix.reshape(_B * _L, _S)
    rowidx = (b_ids * _L + i_ids).astype(jnp.int32).reshape(_NW, _NCHUNK, _CH)
    jr = j_ids.astype(jnp.int32).reshape(_NW, _RPW)
    perm = jax.random.permutation(jax.random.key(42), 2 * _N_NEG)[:_N_NEG]
    masks = (jnp.arange(2 * _LANES)[None, :] == perm[:, None]).astype(
        jnp.float32).sum(axis=0).reshape(2, _LANES)
    out = _triplet_sc(table, rowidx, jr, masks)
    return jnp.sum(out)
